# Initial kernel scaffold; baseline (speedup 1.0000x reference)
import functools

import jax
import jax.numpy as jnp
from jax import lax
from jax.experimental import pallas as pl
from jax.experimental.pallas import tpu as pltpu
from jax.experimental.pallas import tpu_sc as plsc

DIM = 32
C = 128
NBUF = 4


@functools.lru_cache(maxsize=None)
def _build(n_chunks):
    info = plsc.get_sparse_core_info()
    nw = info.num_cores * info.num_subcores
    assert n_chunks % (nw * NBUF) == 0, n_chunks
    cpw = n_chunks // nw
    b = n_chunks * C
    mesh = plsc.VectorSubcoreMesh(core_axis_name="c", subcore_axis_name="s")

    @functools.partial(
        pl.kernel,
        mesh=mesh,
        out_type=jax.ShapeDtypeStruct((b, DIM), jnp.float32),
        scratch_types=(
            [pltpu.VMEM((cpw, C), jnp.int32)]
            + [pltpu.VMEM((C, DIM), jnp.float32) for _ in range(NBUF)]
            + [pltpu.SemaphoreType.DMA for _ in range(2 * NBUF)]
        ),
    )
    def k(table, idx_hbm, out, idx_v, *rest):
        bufs = rest[:NBUF]
        gsems = rest[NBUF:2 * NBUF]
        wsems = rest[2 * NBUF:3 * NBUF]
        wid = lax.axis_index("s") * info.num_cores + lax.axis_index("c")
        base_chunk = wid * cpw
        base_row = base_chunk * C

        pltpu.sync_copy(idx_hbm.at[pl.ds(base_chunk, cpw)], idx_v)

        for buf_i in range(NBUF):
            pltpu.make_async_copy(
                table.at[idx_v.at[buf_i]], bufs[buf_i], gsems[buf_i]
            ).start()

        n_outer = cpw // NBUF

        def outer(jo, carry):
            for buf_i in range(NBUF):
                i = jo * NBUF + buf_i
                row0 = base_row + i * C
                pltpu.make_async_copy(
                    table.at[idx_v.at[i]], bufs[buf_i], gsems[buf_i]
                ).wait()
                pltpu.make_async_copy(
                    bufs[buf_i], out.at[pl.ds(row0, C)], wsems[buf_i]
                ).start()

                @pl.when(jo < n_outer - 1)
                def _():
                    pltpu.make_async_copy(
                        bufs[buf_i], out.at[pl.ds(row0, C)], wsems[buf_i]
                    ).wait()
                    pltpu.make_async_copy(
                        table.at[idx_v.at[i + NBUF]], bufs[buf_i], gsems[buf_i]
                    ).start()

            return carry

        lax.fori_loop(0, n_outer, outer, 0)

        for buf_i in range(NBUF):
            i = cpw - NBUF + buf_i
            row0 = base_row + i * C
            pltpu.make_async_copy(
                bufs[buf_i], out.at[pl.ds(row0, C)], wsems[buf_i]
            ).wait()

    return k


def kernel(ids, emb):
    b = ids.size
    assert b % C == 0, b
    ids2d = ids.reshape(b // C, C).astype(jnp.int32)
    out = _build(b // C)(emb, ids2d)
    return out.reshape(*ids.shape, DIM)


# speedup vs baseline: 1.1107x; 1.1107x over previous
import functools

import jax
import jax.numpy as jnp
from jax import lax
from jax.experimental import pallas as pl
from jax.experimental.pallas import tpu as pltpu
from jax.experimental.pallas import tpu_sc as plsc

DIM = 32
C = 128
NBUF = 4


@functools.lru_cache(maxsize=None)
def _build(n_chunks):
    info = plsc.get_sparse_core_info()
    nw = info.num_cores * info.num_subcores
    assert n_chunks % (nw * NBUF) == 0, n_chunks
    cpw = n_chunks // nw
    b = n_chunks * C
    mesh = plsc.VectorSubcoreMesh(core_axis_name="c", subcore_axis_name="s")

    @functools.partial(
        pl.kernel,
        mesh=mesh,
        out_type=jax.ShapeDtypeStruct((b, DIM), jnp.float32),
        scratch_types=(
            [pltpu.VMEM((cpw, C), jnp.int32)]
            + [pltpu.VMEM((C, DIM), jnp.float32) for _ in range(NBUF)]
            + [pltpu.SemaphoreType.DMA for _ in range(2 * NBUF)]
        ),
        compiler_params=pltpu.CompilerParams(use_tc_tiling_on_sc=False),
    )
    def k(table, idx_hbm, out, idx_v, *rest):
        bufs = rest[:NBUF]
        gsems = rest[NBUF:2 * NBUF]
        wsems = rest[2 * NBUF:3 * NBUF]
        wid = lax.axis_index("s") * info.num_cores + lax.axis_index("c")
        base_chunk = wid * cpw
        base_row = base_chunk * C

        pltpu.sync_copy(idx_hbm.at[pl.ds(base_chunk, cpw)], idx_v)

        for buf_i in range(NBUF):
            pltpu.make_async_copy(
                table.at[idx_v.at[buf_i]], bufs[buf_i], gsems[buf_i]
            ).start()

        n_outer = cpw // NBUF

        def outer(jo, carry):
            for buf_i in range(NBUF):
                i = jo * NBUF + buf_i
                row0 = base_row + i * C
                pltpu.make_async_copy(
                    table.at[idx_v.at[i]], bufs[buf_i], gsems[buf_i]
                ).wait()
                pltpu.make_async_copy(
                    bufs[buf_i], out.at[pl.ds(row0, C)], wsems[buf_i]
                ).start()

                @pl.when(jo < n_outer - 1)
                def _():
                    pltpu.make_async_copy(
                        bufs[buf_i], out.at[pl.ds(row0, C)], wsems[buf_i]
                    ).wait()
                    pltpu.make_async_copy(
                        table.at[idx_v.at[i + NBUF]], bufs[buf_i], gsems[buf_i]
                    ).start()

            return carry

        lax.fori_loop(0, n_outer, outer, 0)

        for buf_i in range(NBUF):
            i = cpw - NBUF + buf_i
            row0 = base_row + i * C
            pltpu.make_async_copy(
                bufs[buf_i], out.at[pl.ds(row0, C)], wsems[buf_i]
            ).wait()

    return k


def kernel(ids, emb):
    b = ids.size
    assert b % C == 0, b
    ids2d = ids.reshape(b // C, C).astype(jnp.int32)
    out = _build(b // C)(emb, ids2d)
    return out.reshape(*ids.shape, DIM)


# speedup vs baseline: 1.1144x; 1.0034x over previous
import functools

import jax
import jax.numpy as jnp
from jax import lax
from jax.experimental import pallas as pl
from jax.experimental.pallas import tpu as pltpu
from jax.experimental.pallas import tpu_sc as plsc

DIM = 32
C = 128
NBUF = 10
LEAD = 8


@functools.lru_cache(maxsize=None)
def _build(n_chunks):
    info = plsc.get_sparse_core_info()
    nw = info.num_cores * info.num_subcores
    assert n_chunks % (nw * NBUF) == 0, n_chunks
    cpw = n_chunks // nw
    b = n_chunks * C
    mesh = plsc.VectorSubcoreMesh(core_axis_name="c", subcore_axis_name="s")

    @functools.partial(
        pl.kernel,
        mesh=mesh,
        out_type=jax.ShapeDtypeStruct((b, DIM), jnp.float32),
        scratch_types=(
            [pltpu.VMEM((cpw, C), jnp.int32)]
            + [pltpu.VMEM((C, DIM), jnp.float32) for _ in range(NBUF)]
            + [pltpu.SemaphoreType.DMA for _ in range(2 * NBUF)]
        ),
        compiler_params=pltpu.CompilerParams(use_tc_tiling_on_sc=False),
    )
    def k(table, idx_hbm, out, idx_v, *rest):
        bufs = rest[:NBUF]
        gsems = rest[NBUF:2 * NBUF]
        wsems = rest[2 * NBUF:3 * NBUF]
        wid = lax.axis_index("s") * info.num_cores + lax.axis_index("c")
        base_chunk = wid * cpw
        base_row = base_chunk * C

        pltpu.sync_copy(idx_hbm.at[pl.ds(base_chunk, cpw)], idx_v)

        for j in range(LEAD):
            pltpu.make_async_copy(
                table.at[idx_v.at[j]], bufs[j % NBUF], gsems[j % NBUF]
            ).start()

        n_outer = cpw // NBUF

        def outer(jo, carry):
            for buf_i in range(NBUF):
                i = jo * NBUF + buf_i
                row0 = base_row + i * C
                pltpu.make_async_copy(
                    table.at[idx_v.at[i]], bufs[buf_i], gsems[buf_i]
                ).wait()
                pltpu.make_async_copy(
                    bufs[buf_i], out.at[pl.ds(row0, C)], wsems[buf_i]
                ).start()
                nb = (buf_i + LEAD) % NBUF

                @pl.when(i + LEAD - NBUF >= 0)
                def _():
                    pltpu.make_async_copy(
                        bufs[nb],
                        out.at[pl.ds(base_row + (i + LEAD - NBUF) * C, C)],
                        wsems[nb],
                    ).wait()

                @pl.when(i + LEAD < cpw)
                def _():
                    pltpu.make_async_copy(
                        table.at[idx_v.at[i + LEAD]], bufs[nb], gsems[nb]
                    ).start()

            return carry

        lax.fori_loop(0, n_outer, outer, 0)

        for t in range(NBUF - LEAD):
            i = cpw - (NBUF - LEAD) + t
            row0 = base_row + i * C
            pltpu.make_async_copy(
                bufs[i % NBUF], out.at[pl.ds(row0, C)], wsems[i % NBUF]
            ).wait()

    return k


def kernel(ids, emb):
    b = ids.size
    assert b % C == 0, b
    ids2d = ids.reshape(b // C, C).astype(jnp.int32)
    out = _build(b // C)(emb, ids2d)
    return out.reshape(*ids.shape, DIM)


# speedup vs baseline: 1.7831x; 1.6001x over previous
import functools

import jax
import jax.numpy as jnp
from jax import lax
from jax.experimental import pallas as pl
from jax.experimental.pallas import tpu as pltpu
from jax.experimental.pallas import tpu_sc as plsc

DIM = 32
SEQ = 50
NBUF = 8
LEAD = 6


@functools.lru_cache(maxsize=None)
def _build(n_samples, vocab):
    info = plsc.get_sparse_core_info()
    nw = info.num_cores * info.num_subcores
    assert n_samples % (nw * NBUF) == 0, n_samples
    cpw = n_samples // nw
    mesh = plsc.VectorSubcoreMesh(core_axis_name="c", subcore_axis_name="s")

    @functools.partial(
        pl.kernel,
        mesh=mesh,
        out_type=jax.ShapeDtypeStruct((n_samples, SEQ, DIM), jnp.float32),
        scratch_types=(
            [pltpu.VMEM((cpw, SEQ), jnp.int32)]
            + [pltpu.VMEM((SEQ, DIM), jnp.float32) for _ in range(NBUF)]
            + [pltpu.SemaphoreType.DMA for _ in range(2 * NBUF)]
        ),
        compiler_params=pltpu.CompilerParams(use_tc_tiling_on_sc=False),
    )
    def k(table, idx_hbm, out, idx_v, *rest):
        bufs = rest[:NBUF]
        gsems = rest[NBUF:2 * NBUF]
        wsems = rest[2 * NBUF:3 * NBUF]
        wid = lax.axis_index("s") * info.num_cores + lax.axis_index("c")
        base = wid * cpw

        pltpu.sync_copy(idx_hbm.at[pl.ds(base, cpw)], idx_v)

        for j in range(LEAD):
            pltpu.make_async_copy(
                table.at[idx_v.at[j]], bufs[j % NBUF], gsems[j % NBUF]
            ).start()

        n_outer = cpw // NBUF

        def outer(jo, carry):
            for buf_i in range(NBUF):
                i = jo * NBUF + buf_i
                pltpu.make_async_copy(
                    table.at[idx_v.at[i]], bufs[buf_i], gsems[buf_i]
                ).wait()
                pltpu.make_async_copy(
                    bufs[buf_i], out.at[base + i], wsems[buf_i]
                ).start()
                nb = (buf_i + LEAD) % NBUF

                @pl.when(i + LEAD - NBUF >= 0)
                def _():
                    pltpu.make_async_copy(
                        bufs[nb],
                        out.at[base + i + LEAD - NBUF],
                        wsems[nb],
                    ).wait()

                @pl.when(i + LEAD < cpw)
                def _():
                    pltpu.make_async_copy(
                        table.at[idx_v.at[i + LEAD]], bufs[nb], gsems[nb]
                    ).start()

            return carry

        lax.fori_loop(0, n_outer, outer, 0)

        for t in range(NBUF - LEAD):
            i = cpw - (NBUF - LEAD) + t
            pltpu.make_async_copy(
                bufs[i % NBUF], out.at[base + i], wsems[i % NBUF]
            ).wait()

    return k


def kernel(ids, emb):
    n_samples, seq = ids.shape
    assert seq == SEQ and emb.shape[1] == DIM, (ids.shape, emb.shape)
    return _build(n_samples, emb.shape[0])(emb, ids.astype(jnp.int32))
